# baseline (speedup 1.0000x reference)
import jax
import jax.numpy as jnp
import numpy as np
from jax.experimental import pallas as pl

N_NODES = 10000
RATIO = 0.5


def _mlp_kernel(xsum_ref, w1_ref, b1_ref, w2_ref, b2_ref, w3_ref, b3_ref, logits_ref):
    h = jnp.maximum(xsum_ref[...] @ w1_ref[...] + b1_ref[...], 0.0)
    h = jnp.maximum(h @ w2_ref[...] + b2_ref[...], 0.0)
    logits_ref[...] = h @ w3_ref[...] + b3_ref[...]


def _gcn_conv(x, row, col, emask, n, W, b):
    h = x @ W
    sl = jnp.arange(n, dtype=row.dtype)
    r = jnp.concatenate([row, sl])
    c = jnp.concatenate([col, sl])
    w = jnp.concatenate([emask, jnp.ones((n,), x.dtype)])
    deg = jnp.zeros((n,), x.dtype).at[c].add(w)
    dinv = jnp.where(deg > 0, 1.0 / jnp.sqrt(jnp.where(deg > 0, deg, 1.0)), 0.0)
    norm = dinv[r] * w * dinv[c]
    out = jnp.zeros((n, h.shape[1]), x.dtype).at[c].add(norm[:, None] * h[r])
    return out + b


def _graph_conv_score(x, row, col, emask, n, Wrel, brel, Wroot):
    agg = jnp.zeros((n, x.shape[1]), x.dtype).at[col].add(emask[:, None] * x[row])
    return (agg @ Wrel + brel + x @ Wroot).reshape(-1)


def _sag_pool(x, row, col, emask, n, k, Wrel, brel, Wroot):
    attn = _graph_conv_score(x, row, col, emask, n, Wrel, brel, Wroot)
    score = jnp.tanh(attn)
    vals, perm = jax.lax.top_k(score, k)
    x_new = x[perm] * vals[:, None]
    newidx = jnp.full((n,), -1, jnp.int32).at[perm].set(jnp.arange(k, dtype=jnp.int32))
    r2 = newidx[row]
    c2 = newidx[col]
    valid = (r2 >= 0) & (c2 >= 0) & (emask > 0)
    row_new = jnp.where(valid, r2, 0)
    col_new = jnp.where(valid, c2, 0)
    emask_new = valid.astype(x.dtype)
    return x_new, row_new, col_new, emask_new, perm, vals


def _normalize_scores(s):
    mn = s.min(); mx = s.max()
    denom = jnp.where(mx > mn, mx - mn, 1.0)
    return jnp.where(mx > mn, (s - mn) / denom, jnp.ones_like(s) * 0.5)


def kernel(x, edge_index, batch, params):
    row = edge_index[0].astype(jnp.int32)
    col = edge_index[1].astype(jnp.int32)
    emask = jnp.ones((row.shape[0],), x.dtype)
    n_files = N_NODES
    per_layer = []
    cum = jnp.zeros((n_files,), x.dtype)
    xs = []
    k = x.shape[0]
    for i in range(1, 5):
        h = _gcn_conv(x, row, col, emask, k, params['conv%d_W' % i], params['conv%d_b' % i])
        h = jax.nn.relu(h)
        k_next = int(np.ceil(RATIO * k))
        x, row, col, emask, perm, vals = _sag_pool(h, row, col, emask, k, k_next,
                                                   params['pool%d_Wrel' % i], params['pool%d_brel' % i], params['pool%d_Wroot' % i])
        k = k_next
        xi = jnp.concatenate([x.mean(axis=0), x.max(axis=0)])[None, :]
        xs.append(xi)
        layer_s = jnp.zeros((n_files,), x.dtype).at[perm].set(vals)
        per_layer.append(_normalize_scores(layer_s))
        cum = cum.at[perm].add(vals)
    xsum = xs[0] + xs[1] + xs[2] + xs[3]
    logits = pl.pallas_call(
        _mlp_kernel,
        out_shape=jax.ShapeDtypeStruct((1, 10), jnp.float32),
    )(xsum, params['lin1_W'], params['lin1_b'][None, :],
      params['lin2_W'], params['lin2_b'][None, :],
      params['lin3_W'], params['lin3_b'][None, :])
    probs = jax.nn.softmax(logits, axis=1)
    return logits, probs, jnp.stack(per_layer), _normalize_scores(cum)


# speedup vs baseline: 2.0275x; 2.0275x over previous
import functools
import jax
import jax.numpy as jnp
import numpy as np
from jax import lax
from jax.experimental import pallas as pl
from jax.experimental.pallas import tpu as pltpu, tpu_sc as plsc

N_NODES = 10000
RATIO = 0.5

NW = 32
BATCH = 64
CHUNK = 2048


def _fold_body(kp, ep, weighted, *refs):
    if weighted:
        (z_hbm, dinv_hbm, row_hbm, col_hbm, em_hbm, agg_hbm,
         colb, emb, rowb, pend_g, pend_l, gbuf, acc, dinvb, drb, sem_e, sem_g) = refs
    else:
        (z_hbm, row_hbm, col_hbm, em_hbm, agg_hbm,
         colb, emb, rowb, pend_g, pend_l, gbuf, acc, sem_e, sem_g) = refs
    R = kp // NW
    cid = lax.axis_index("c")
    sid = lax.axis_index("s")
    wid = sid * 2 + cid
    base = wid * R

    zero16 = jnp.zeros((16,), jnp.float32)

    def _z(r, _):
        for gg in range(16):
            acc[r, pl.ds(gg * 16, 16)] = zero16
        return 0
    lax.fori_loop(0, R + 1, _z, 0)
    if weighted:
        pltpu.sync_copy(dinv_hbm.at[pl.ds(base, R)], dinvb.at[pl.ds(0, R)])

    def add_batch(n):
        def add_row(j, _):
            l = pend_l[pl.ds(j, 16)][0]
            if weighted:
                dr = drb[pl.ds(j, 16)][0]
                dc = dinvb[pl.ds(l, 16)][0]
                w16 = lax.broadcast(dr * dc, (16,))
                for gg in range(16):
                    plsc.addupdate(acc.at[l, pl.ds(gg * 16, 16)],
                                   gbuf[j, pl.ds(gg * 16, 16)] * w16)
            else:
                for gg in range(16):
                    plsc.addupdate(acc.at[l, pl.ds(gg * 16, 16)],
                                   gbuf[j, pl.ds(gg * 16, 16)])
            return 0
        lax.fori_loop(0, n, add_row, 0)

    def fire(cnt):
        pltpu.async_copy(z_hbm.at[pend_g.at[pl.ds(0, BATCH)]], gbuf, sem_g).wait()
        if weighted:
            pltpu.async_copy(dinv_hbm.at[pend_g.at[pl.ds(0, BATCH)]], drb.at[pl.ds(0, BATCH)], sem_g).wait()
        add_batch(BATCH)
        pend_g[pl.ds(0, 16)] = pend_g[pl.ds(BATCH, 16)]
        pend_l[pl.ds(0, 16)] = pend_l[pl.ds(BATCH, 16)]
        return cnt - BATCH

    def chunk_body(ci, cnt):
        e0 = ci * CHUNK
        pltpu.async_copy(col_hbm.at[pl.ds(e0, CHUNK)], colb, sem_e).wait()
        pltpu.async_copy(em_hbm.at[pl.ds(e0, CHUNK)], emb, sem_e).wait()
        pltpu.async_copy(row_hbm.at[pl.ds(e0, CHUNK)], rowb, sem_e).wait()

        def grp_body(g, cnt):
            cvec = colb[pl.ds(g * 16, 16)]
            evec = emb[pl.ds(g * 16, 16)]
            rvec = rowb[pl.ds(g * 16, 16)]
            pred = (cvec >= base) & (cvec < base + R) & (evec > 0.0)
            lane = lax.broadcasted_iota(jnp.int32, (16,), 0)
            key = jnp.where(pred, lane, jnp.full((16,), 999, jnp.int32))
            _, sr = plsc.sort_key_val(key, rvec)
            _, sc2 = plsc.sort_key_val(key, cvec - base)
            pend_g[pl.ds(cnt, 16)] = sr
            pend_l[pl.ds(cnt, 16)] = sc2
            cnt = cnt + plsc.all_reduce_population_count(pred)[0]
            return lax.cond(cnt >= BATCH, fire, lambda c: c, cnt)

        return lax.fori_loop(0, CHUNK // 16, grp_body, cnt)

    cnt = lax.fori_loop(0, ep // CHUNK, chunk_body, jnp.int32(0))

    padg = jnp.full((16,), base, jnp.int32)
    padl = jnp.full((16,), R, jnp.int32)
    for j in range(BATCH // 16):
        pend_g[pl.ds(cnt + j * 16, 16)] = padg
        pend_l[pl.ds(cnt + j * 16, 16)] = padl
    pltpu.async_copy(z_hbm.at[pend_g.at[pl.ds(0, BATCH)]], gbuf, sem_g).wait()
    if weighted:
        pltpu.async_copy(dinv_hbm.at[pend_g.at[pl.ds(0, BATCH)]], drb.at[pl.ds(0, BATCH)], sem_g).wait()
    add_batch(cnt)

    pltpu.sync_copy(acc.at[pl.ds(0, R)], agg_hbm.at[pl.ds(base, R)])


def _seg_fold(z, row, col, em, dinv=None, *, kp, ep):
    R = kp // NW
    weighted = dinv is not None
    body = functools.partial(_fold_body, kp, ep, weighted)
    scratch = [
        pltpu.VMEM((CHUNK,), jnp.int32),
        pltpu.VMEM((CHUNK,), jnp.float32),
        pltpu.VMEM((CHUNK,), jnp.int32),
        pltpu.VMEM((BATCH + 32,), jnp.int32),
        pltpu.VMEM((BATCH + 32,), jnp.int32),
        pltpu.VMEM((BATCH, 256), jnp.float32),
        pltpu.VMEM(((R + 1), 256), jnp.float32),
    ]
    if weighted:
        scratch += [pltpu.VMEM((R + 16,), jnp.float32), pltpu.VMEM((BATCH + 16,), jnp.float32)]
    scratch += [pltpu.SemaphoreType.DMA, pltpu.SemaphoreType.DMA]
    args = (z, dinv, row, col, em) if weighted else (z, row, col, em)
    return pl.kernel(
        body,
        out_type=jax.ShapeDtypeStruct((kp, 256), jnp.float32),
        mesh=plsc.VectorSubcoreMesh(core_axis_name="c", subcore_axis_name="s"),
        compiler_params=pltpu.CompilerParams(needs_layout_passes=False),
        scratch_types=scratch,
    )(*args)


def _sc_agg(x, row, col, emask, n, dinv=None):
    kp = (n + 511) // 512 * 512
    E = row.shape[0]
    ep = (E + CHUNK - 1) // CHUNK * CHUNK
    xp = jnp.pad(x, ((0, kp - n), (0, 0)))
    rp = jnp.pad(row, (0, ep - E))
    cp = jnp.pad(col, (0, ep - E))
    em = jnp.pad(emask, (0, ep - E))
    dp = None if dinv is None else jnp.pad(dinv, (0, kp - n))
    agg = _seg_fold(xp, rp, cp, em, dp, kp=kp, ep=ep)
    return agg[:n]


def _mm_kernel(x_ref, w_ref, o_ref):
    o_ref[...] = jnp.dot(x_ref[...], w_ref[...])


def _pallas_mm(x, w):
    n, d = x.shape
    npad = (n + 255) // 256 * 256
    xp = jnp.pad(x, ((0, npad - n), (0, 0)))
    out = pl.pallas_call(
        _mm_kernel,
        grid=(npad // 256,),
        in_specs=[pl.BlockSpec((256, d), lambda i: (i, 0)),
                  pl.BlockSpec((d, w.shape[1]), lambda i: (0, 0))],
        out_specs=pl.BlockSpec((256, w.shape[1]), lambda i: (i, 0)),
        out_shape=jax.ShapeDtypeStruct((npad, w.shape[1]), jnp.float32),
    )(xp, w)
    return out[:n]


def _gcn_conv(x, row, col, emask, n, W, b, use_sc=True):
    h = _pallas_mm(x, W)
    if use_sc:
        sl = jnp.arange(n, dtype=row.dtype)
        c = jnp.concatenate([col, sl])
        w = jnp.concatenate([emask, jnp.ones((n,), x.dtype)])
        deg = jnp.zeros((n,), x.dtype).at[c].add(w)
        dinv = jnp.where(deg > 0, 1.0 / jnp.sqrt(jnp.where(deg > 0, deg, 1.0)), 0.0)
        agg = _sc_agg(h, row, col, emask, n, dinv)
        return (agg + (dinv * dinv)[:, None] * h) + b
    sl = jnp.arange(n, dtype=row.dtype)
    r = jnp.concatenate([row, sl])
    c = jnp.concatenate([col, sl])
    w = jnp.concatenate([emask, jnp.ones((n,), x.dtype)])
    deg = jnp.zeros((n,), x.dtype).at[c].add(w)
    dinv = jnp.where(deg > 0, 1.0 / jnp.sqrt(jnp.where(deg > 0, deg, 1.0)), 0.0)
    norm = dinv[r] * w * dinv[c]
    out = jnp.zeros((n, h.shape[1]), x.dtype).at[c].add(norm[:, None] * h[r])
    return out + b


def _graph_conv_score(x, row, col, emask, n, Wrel, brel, Wroot):
    agg = _sc_agg(x, row, col, emask, n)
    return (agg @ Wrel + brel + x @ Wroot).reshape(-1)


def _sag_pool(x, row, col, emask, n, k, Wrel, brel, Wroot):
    attn = _graph_conv_score(x, row, col, emask, n, Wrel, brel, Wroot)
    score = jnp.tanh(attn)
    vals, perm = jax.lax.top_k(score, k)
    x_new = x[perm] * vals[:, None]
    newidx = jnp.full((n,), -1, jnp.int32).at[perm].set(jnp.arange(k, dtype=jnp.int32))
    r2 = newidx[row]
    c2 = newidx[col]
    valid = (r2 >= 0) & (c2 >= 0) & (emask > 0)
    row_new = jnp.where(valid, r2, 0)
    col_new = jnp.where(valid, c2, 0)
    emask_new = valid.astype(x.dtype)
    return x_new, row_new, col_new, emask_new, perm, vals


def _normalize_scores(s):
    mn = s.min(); mx = s.max()
    denom = jnp.where(mx > mn, mx - mn, 1.0)
    return jnp.where(mx > mn, (s - mn) / denom, jnp.ones_like(s) * 0.5)


def kernel(x, edge_index, batch, params):
    row = edge_index[0].astype(jnp.int32)
    col = edge_index[1].astype(jnp.int32)
    emask = jnp.ones((row.shape[0],), x.dtype)
    n_files = N_NODES
    per_layer = []
    cum = jnp.zeros((n_files,), x.dtype)
    xs = []
    k = x.shape[0]
    for i in range(1, 5):
        h = _gcn_conv(x, row, col, emask, k, params['conv%d_W' % i], params['conv%d_b' % i],
                      use_sc=True)
        h = jax.nn.relu(h)
        k_next = int(np.ceil(RATIO * k))
        x, row, col, emask, perm, vals = _sag_pool(h, row, col, emask, k, k_next,
                                                   params['pool%d_Wrel' % i], params['pool%d_brel' % i], params['pool%d_Wroot' % i])
        k = k_next
        xi = jnp.concatenate([x.mean(axis=0), x.max(axis=0)])[None, :]
        xs.append(xi)
        layer_s = jnp.zeros((n_files,), x.dtype).at[perm].set(vals)
        per_layer.append(_normalize_scores(layer_s))
        cum = cum.at[perm].add(vals)
    xsum = xs[0] + xs[1] + xs[2] + xs[3]
    h = jax.nn.relu(xsum @ params['lin1_W'] + params['lin1_b'])
    h = jax.nn.relu(h @ params['lin2_W'] + params['lin2_b'])
    logits = h @ params['lin3_W'] + params['lin3_b']
    probs = jax.nn.softmax(logits, axis=1)
    return logits, probs, jnp.stack(per_layer), _normalize_scores(cum)


# speedup vs baseline: 2.2131x; 1.0915x over previous
import functools
import jax
import jax.numpy as jnp
import numpy as np
from jax import lax
from jax.experimental import pallas as pl
from jax.experimental.pallas import tpu as pltpu, tpu_sc as plsc

N_NODES = 10000
RATIO = 0.5

NW = 32
BATCH = 128
CHUNK = 1024


def _fold_body(kp, ep, weighted, *refs):
    if weighted:
        (z_hbm, dinv_hbm, epack_hbm, agg_hbm,
         ebuf0, ebuf1, pend_g, pend_l, gbuf, acc, dinvb, drb, sem_e0, sem_e1, sem_g) = refs
    else:
        (z_hbm, epack_hbm, agg_hbm,
         ebuf0, ebuf1, pend_g, pend_l, gbuf, acc, sem_e0, sem_e1, sem_g) = refs
    R = kp // NW
    cid = lax.axis_index("c")
    sid = lax.axis_index("s")
    wid = sid * 2 + cid
    base = wid * R

    zero16 = jnp.zeros((16,), jnp.float32)

    def _z(r, _):
        for gg in range(16):
            acc[r, pl.ds(gg * 16, 16)] = zero16
        return 0
    lax.fori_loop(0, R + 1, _z, 0)
    if weighted:
        pltpu.sync_copy(dinv_hbm.at[pl.ds(base, R)], dinvb.at[pl.ds(0, R)])

    def add_batch(n):
        def add_row(j, _):
            l = pend_l[pl.ds(j, 16)][0]
            if weighted:
                dr = drb[pl.ds(j, 16)][0]
                dc = dinvb[pl.ds(l, 16)][0]
                w16 = lax.broadcast(dr * dc, (16,))
                for gg in range(16):
                    plsc.addupdate(acc.at[l, pl.ds(gg * 16, 16)],
                                   gbuf[j, pl.ds(gg * 16, 16)] * w16)
            else:
                for gg in range(16):
                    plsc.addupdate(acc.at[l, pl.ds(gg * 16, 16)],
                                   gbuf[j, pl.ds(gg * 16, 16)])
            return 0
        lax.fori_loop(0, n, add_row, 0)

    def fire(cnt):
        pltpu.async_copy(z_hbm.at[pend_g.at[pl.ds(0, BATCH)]], gbuf, sem_g).wait()
        if weighted:
            pltpu.async_copy(dinv_hbm.at[pend_g.at[pl.ds(0, BATCH)]], drb.at[pl.ds(0, BATCH)], sem_g).wait()
        add_batch(BATCH)
        pend_g[pl.ds(0, 16)] = pend_g[pl.ds(BATCH, 16)]
        pend_l[pl.ds(0, 16)] = pend_l[pl.ds(BATCH, 16)]
        return cnt - BATCH

    def scan(ebuf, cnt):
        def grp_body(g, cnt):
            rvec = ebuf[pl.ds(g * 16, 16)]
            cvec = ebuf[pl.ds(CHUNK + g * 16, 16)]
            ivec = ebuf[pl.ds(2 * CHUNK + g * 16, 16)]
            pred = (cvec >= base) & (cvec < base + R) & (ivec > 0)
            lane = lax.broadcasted_iota(jnp.int32, (16,), 0)
            key = jnp.where(pred, lane, jnp.full((16,), 999, jnp.int32))
            _, sr = plsc.sort_key_val(key, rvec)
            _, sc2 = plsc.sort_key_val(key, cvec - base)
            pend_g[pl.ds(cnt, 16)] = sr
            pend_l[pl.ds(cnt, 16)] = sc2
            cnt = cnt + plsc.all_reduce_population_count(pred)[0]
            return lax.cond(cnt >= BATCH, fire, lambda c: c, cnt)
        return lax.fori_loop(0, CHUNK // 16, grp_body, cnt)

    nchunks = ep // CHUNK
    C3 = 3 * CHUNK
    pltpu.async_copy(epack_hbm.at[pl.ds(0, C3)], ebuf0, sem_e0)
    pltpu.async_copy(epack_hbm.at[pl.ds(C3, C3)], ebuf1, sem_e1)

    def pair_body(p, cnt):
        c0 = 2 * p
        pltpu.make_async_copy(epack_hbm.at[pl.ds(0, C3)], ebuf0, sem_e0).wait()
        cnt = scan(ebuf0, cnt)
        pltpu.async_copy(epack_hbm.at[pl.ds((c0 + 2) * C3, C3)], ebuf0, sem_e0)
        pltpu.make_async_copy(epack_hbm.at[pl.ds(0, C3)], ebuf1, sem_e1).wait()
        cnt = scan(ebuf1, cnt)
        pltpu.async_copy(epack_hbm.at[pl.ds((c0 + 3) * C3, C3)], ebuf1, sem_e1)
        return cnt

    cnt = lax.fori_loop(0, nchunks // 2, pair_body, jnp.int32(0))
    pltpu.make_async_copy(epack_hbm.at[pl.ds(0, C3)], ebuf0, sem_e0).wait()
    pltpu.make_async_copy(epack_hbm.at[pl.ds(0, C3)], ebuf1, sem_e1).wait()

    padg = jnp.full((16,), base, jnp.int32)
    padl = jnp.full((16,), R, jnp.int32)
    for j in range(BATCH // 16):
        pend_g[pl.ds(cnt + j * 16, 16)] = padg
        pend_l[pl.ds(cnt + j * 16, 16)] = padl
    pltpu.async_copy(z_hbm.at[pend_g.at[pl.ds(0, BATCH)]], gbuf, sem_g).wait()
    if weighted:
        pltpu.async_copy(dinv_hbm.at[pend_g.at[pl.ds(0, BATCH)]], drb.at[pl.ds(0, BATCH)], sem_g).wait()
    add_batch(cnt)

    pltpu.sync_copy(acc.at[pl.ds(0, R)], agg_hbm.at[pl.ds(base, R)])


def _seg_fold(z, row, col, em, dinv=None, *, kp, ep):
    R = kp // NW
    weighted = dinv is not None
    body = functools.partial(_fold_body, kp, ep, weighted)
    scratch = [
        pltpu.VMEM((3 * CHUNK,), jnp.int32),
        pltpu.VMEM((3 * CHUNK,), jnp.int32),
        pltpu.VMEM((BATCH + 32,), jnp.int32),
        pltpu.VMEM((BATCH + 32,), jnp.int32),
        pltpu.VMEM((BATCH, 256), jnp.float32),
        pltpu.VMEM(((R + 1), 256), jnp.float32),
    ]
    if weighted:
        scratch += [pltpu.VMEM((R + 16,), jnp.float32), pltpu.VMEM((BATCH + 16,), jnp.float32)]
    scratch += [pltpu.SemaphoreType.DMA, pltpu.SemaphoreType.DMA, pltpu.SemaphoreType.DMA]
    nchunks = ep // CHUNK
    epack = jnp.stack([row.reshape(nchunks, CHUNK),
                       col.reshape(nchunks, CHUNK),
                       jax.lax.bitcast_convert_type(em, jnp.int32).reshape(nchunks, CHUNK)],
                      axis=1).reshape(-1)
    epack = jnp.concatenate([epack, jnp.zeros((2 * 3 * CHUNK,), jnp.int32)])
    args = (z, dinv, epack) if weighted else (z, epack)
    return pl.kernel(
        body,
        out_type=jax.ShapeDtypeStruct((kp, 256), jnp.float32),
        mesh=plsc.VectorSubcoreMesh(core_axis_name="c", subcore_axis_name="s"),
        compiler_params=pltpu.CompilerParams(needs_layout_passes=False),
        scratch_types=scratch,
    )(*args)


def _sc_agg(x, row, col, emask, n, dinv=None):
    kp = (n + 511) // 512 * 512
    E = row.shape[0]
    ep = (E + 2 * CHUNK - 1) // (2 * CHUNK) * (2 * CHUNK)
    xp = jnp.pad(x, ((0, kp - n), (0, 0)))
    rp = jnp.pad(row, (0, ep - E))
    cp = jnp.pad(col, (0, ep - E))
    em = jnp.pad(emask, (0, ep - E))
    dp = None if dinv is None else jnp.pad(dinv, (0, kp - n))
    agg = _seg_fold(xp, rp, cp, em, dp, kp=kp, ep=ep)
    return agg[:n]


def _mm_kernel(x_ref, w_ref, o_ref):
    o_ref[...] = jnp.dot(x_ref[...], w_ref[...])


def _pallas_mm(x, w):
    n, d = x.shape
    npad = (n + 255) // 256 * 256
    xp = jnp.pad(x, ((0, npad - n), (0, 0)))
    out = pl.pallas_call(
        _mm_kernel,
        grid=(npad // 256,),
        in_specs=[pl.BlockSpec((256, d), lambda i: (i, 0)),
                  pl.BlockSpec((d, w.shape[1]), lambda i: (0, 0))],
        out_specs=pl.BlockSpec((256, w.shape[1]), lambda i: (i, 0)),
        out_shape=jax.ShapeDtypeStruct((npad, w.shape[1]), jnp.float32),
    )(xp, w)
    return out[:n]


def _gcn_conv(x, row, col, emask, n, W, b, use_sc=True):
    h = _pallas_mm(x, W)
    if use_sc:
        sl = jnp.arange(n, dtype=row.dtype)
        c = jnp.concatenate([col, sl])
        w = jnp.concatenate([emask, jnp.ones((n,), x.dtype)])
        deg = jnp.zeros((n,), x.dtype).at[c].add(w)
        dinv = jnp.where(deg > 0, 1.0 / jnp.sqrt(jnp.where(deg > 0, deg, 1.0)), 0.0)
        agg = _sc_agg(h, row, col, emask, n, dinv)
        return (agg + (dinv * dinv)[:, None] * h) + b
    sl = jnp.arange(n, dtype=row.dtype)
    r = jnp.concatenate([row, sl])
    c = jnp.concatenate([col, sl])
    w = jnp.concatenate([emask, jnp.ones((n,), x.dtype)])
    deg = jnp.zeros((n,), x.dtype).at[c].add(w)
    dinv = jnp.where(deg > 0, 1.0 / jnp.sqrt(jnp.where(deg > 0, deg, 1.0)), 0.0)
    norm = dinv[r] * w * dinv[c]
    out = jnp.zeros((n, h.shape[1]), x.dtype).at[c].add(norm[:, None] * h[r])
    return out + b


def _graph_conv_score(x, row, col, emask, n, Wrel, brel, Wroot):
    agg = _sc_agg(x, row, col, emask, n)
    return (agg @ Wrel + brel + x @ Wroot).reshape(-1)


def _sag_pool(x, row, col, emask, n, k, Wrel, brel, Wroot):
    attn = _graph_conv_score(x, row, col, emask, n, Wrel, brel, Wroot)
    score = jnp.tanh(attn)
    vals, perm = jax.lax.top_k(score, k)
    x_new = x[perm] * vals[:, None]
    newidx = jnp.full((n,), -1, jnp.int32).at[perm].set(jnp.arange(k, dtype=jnp.int32))
    r2 = newidx[row]
    c2 = newidx[col]
    valid = (r2 >= 0) & (c2 >= 0) & (emask > 0)
    row_new = jnp.where(valid, r2, 0)
    col_new = jnp.where(valid, c2, 0)
    emask_new = valid.astype(x.dtype)
    return x_new, row_new, col_new, emask_new, perm, vals


def _normalize_scores(s):
    mn = s.min(); mx = s.max()
    denom = jnp.where(mx > mn, mx - mn, 1.0)
    return jnp.where(mx > mn, (s - mn) / denom, jnp.ones_like(s) * 0.5)


def kernel(x, edge_index, batch, params):
    row = edge_index[0].astype(jnp.int32)
    col = edge_index[1].astype(jnp.int32)
    emask = jnp.ones((row.shape[0],), x.dtype)
    n_files = N_NODES
    per_layer = []
    cum = jnp.zeros((n_files,), x.dtype)
    xs = []
    k = x.shape[0]
    for i in range(1, 5):
        h = _gcn_conv(x, row, col, emask, k, params['conv%d_W' % i], params['conv%d_b' % i],
                      use_sc=True)
        h = jax.nn.relu(h)
        k_next = int(np.ceil(RATIO * k))
        x, row, col, emask, perm, vals = _sag_pool(h, row, col, emask, k, k_next,
                                                   params['pool%d_Wrel' % i], params['pool%d_brel' % i], params['pool%d_Wroot' % i])
        k = k_next
        xi = jnp.concatenate([x.mean(axis=0), x.max(axis=0)])[None, :]
        xs.append(xi)
        layer_s = jnp.zeros((n_files,), x.dtype).at[perm].set(vals)
        per_layer.append(_normalize_scores(layer_s))
        cum = cum.at[perm].add(vals)
    xsum = xs[0] + xs[1] + xs[2] + xs[3]
    h = jax.nn.relu(xsum @ params['lin1_W'] + params['lin1_b'])
    h = jax.nn.relu(h @ params['lin2_W'] + params['lin2_b'])
    logits = h @ params['lin3_W'] + params['lin3_b']
    probs = jax.nn.softmax(logits, axis=1)
    return logits, probs, jnp.stack(per_layer), _normalize_scores(cum)


# speedup vs baseline: 2.2198x; 1.0030x over previous
import functools
import jax
import jax.numpy as jnp
import numpy as np
from jax import lax
from jax.experimental import pallas as pl
from jax.experimental.pallas import tpu as pltpu, tpu_sc as plsc

N_NODES = 10000
RATIO = 0.5

NW = 32
BATCH = 128
CHUNK = 1024


def _fold_body(kp, ep, weighted, *refs):
    if weighted:
        (z_hbm, dinv_hbm, epack_hbm, agg_hbm,
         ebuf0, ebuf1, pend_p, pend_g, pend_l, gbuf, acc, dinvb, drb,
         sem_e0, sem_e1, sem_g, sem_d) = refs
    else:
        (z_hbm, epack_hbm, agg_hbm,
         ebuf0, ebuf1, pend_p, pend_g, pend_l, gbuf, acc,
         sem_e0, sem_e1, sem_g, sem_d) = refs
    R = kp // NW
    cid = lax.axis_index("c")
    sid = lax.axis_index("s")
    wid = sid * 2 + cid
    base = wid * R

    zero16 = jnp.zeros((16,), jnp.float32)

    def _z(r, _):
        for gg in range(16):
            acc[r, pl.ds(gg * 16, 16)] = zero16
        return 0
    lax.fori_loop(0, R + 1, _z, 0)
    if weighted:
        pltpu.sync_copy(dinv_hbm.at[pl.ds(base, R)], dinvb.at[pl.ds(0, R)])

    def add_batch(n):
        def add_row(j, _):
            l = pend_l[pl.ds(j, 16)][0]
            if weighted:
                dr = drb[pl.ds(j, 16)][0]
                dc = dinvb[pl.ds(l, 16)][0]
                w16 = lax.broadcast(dr * dc, (16,))
                for gg in range(16):
                    plsc.addupdate(acc.at[l, pl.ds(gg * 16, 16)],
                                   gbuf[j, pl.ds(gg * 16, 16)] * w16)
            else:
                for gg in range(16):
                    plsc.addupdate(acc.at[l, pl.ds(gg * 16, 16)],
                                   gbuf[j, pl.ds(gg * 16, 16)])
            return 0
        lax.fori_loop(0, n, add_row, 0)

    def unpack_pend():
        for gg in range(BATCH // 16):
            v = pend_p[pl.ds(gg * 16, 16)]
            pend_g[pl.ds(gg * 16, 16)] = lax.shift_right_logical(v, 9)
            pend_l[pl.ds(gg * 16, 16)] = v & 511

    def fire(cnt):
        unpack_pend()
        h1 = pltpu.async_copy(z_hbm.at[pend_g.at[pl.ds(0, BATCH)]], gbuf, sem_g)
        if weighted:
            h2 = pltpu.async_copy(dinv_hbm.at[pend_g.at[pl.ds(0, BATCH)]], drb.at[pl.ds(0, BATCH)], sem_d)
        h1.wait()
        if weighted:
            h2.wait()
        add_batch(BATCH)
        pend_p[pl.ds(0, 16)] = pend_p[pl.ds(BATCH, 16)]
        return cnt - BATCH

    def scan(ebuf, cnt):
        def grp_body(g, cnt):
            rvec = ebuf[pl.ds(g * 16, 16)]
            cvec = ebuf[pl.ds(CHUNK + g * 16, 16)]
            ivec = ebuf[pl.ds(2 * CHUNK + g * 16, 16)]
            pred = (cvec >= base) & (cvec < base + R) & (ivec > 0)
            lane = lax.broadcasted_iota(jnp.int32, (16,), 0)
            key = jnp.where(pred, lane, jnp.full((16,), 999, jnp.int32))
            packed = lax.shift_left(rvec, 9) | (cvec - base)
            _, sp = plsc.sort_key_val(key, packed)
            pend_p[pl.ds(cnt, 16)] = sp
            cnt = cnt + plsc.all_reduce_population_count(pred)[0]
            return lax.cond(cnt >= BATCH, fire, lambda c: c, cnt)
        return lax.fori_loop(0, CHUNK // 16, grp_body, cnt)

    nchunks = ep // CHUNK
    C3 = 3 * CHUNK
    pltpu.async_copy(epack_hbm.at[pl.ds(0, C3)], ebuf0, sem_e0)
    pltpu.async_copy(epack_hbm.at[pl.ds(C3, C3)], ebuf1, sem_e1)

    def pair_body(p, cnt):
        c0 = 2 * p
        pltpu.make_async_copy(epack_hbm.at[pl.ds(0, C3)], ebuf0, sem_e0).wait()
        cnt = scan(ebuf0, cnt)
        pltpu.async_copy(epack_hbm.at[pl.ds((c0 + 2) * C3, C3)], ebuf0, sem_e0)
        pltpu.make_async_copy(epack_hbm.at[pl.ds(0, C3)], ebuf1, sem_e1).wait()
        cnt = scan(ebuf1, cnt)
        pltpu.async_copy(epack_hbm.at[pl.ds((c0 + 3) * C3, C3)], ebuf1, sem_e1)
        return cnt

    cnt = lax.fori_loop(0, nchunks // 2, pair_body, jnp.int32(0))
    pltpu.make_async_copy(epack_hbm.at[pl.ds(0, C3)], ebuf0, sem_e0).wait()
    pltpu.make_async_copy(epack_hbm.at[pl.ds(0, C3)], ebuf1, sem_e1).wait()

    padp = jnp.full((16,), base * 512 + R, jnp.int32)
    for j in range(BATCH // 16):
        pend_p[pl.ds(cnt + j * 16, 16)] = padp
    unpack_pend()
    h1 = pltpu.async_copy(z_hbm.at[pend_g.at[pl.ds(0, BATCH)]], gbuf, sem_g)
    if weighted:
        h2 = pltpu.async_copy(dinv_hbm.at[pend_g.at[pl.ds(0, BATCH)]], drb.at[pl.ds(0, BATCH)], sem_d)
    h1.wait()
    if weighted:
        h2.wait()
    add_batch(cnt)

    pltpu.sync_copy(acc.at[pl.ds(0, R)], agg_hbm.at[pl.ds(base, R)])


def _seg_fold(z, row, col, em, dinv=None, *, kp, ep):
    R = kp // NW
    weighted = dinv is not None
    body = functools.partial(_fold_body, kp, ep, weighted)
    scratch = [
        pltpu.VMEM((3 * CHUNK,), jnp.int32),
        pltpu.VMEM((3 * CHUNK,), jnp.int32),
        pltpu.VMEM((BATCH + 32,), jnp.int32),
        pltpu.VMEM((BATCH + 16,), jnp.int32),
        pltpu.VMEM((BATCH + 16,), jnp.int32),
        pltpu.VMEM((BATCH, 256), jnp.float32),
        pltpu.VMEM(((R + 1), 256), jnp.float32),
    ]
    if weighted:
        scratch += [pltpu.VMEM((R + 16,), jnp.float32), pltpu.VMEM((BATCH + 16,), jnp.float32)]
    scratch += [pltpu.SemaphoreType.DMA, pltpu.SemaphoreType.DMA,
                pltpu.SemaphoreType.DMA, pltpu.SemaphoreType.DMA]
    nchunks = ep // CHUNK
    epack = jnp.stack([row.reshape(nchunks, CHUNK),
                       col.reshape(nchunks, CHUNK),
                       jax.lax.bitcast_convert_type(em, jnp.int32).reshape(nchunks, CHUNK)],
                      axis=1).reshape(-1)
    epack = jnp.concatenate([epack, jnp.zeros((2 * 3 * CHUNK,), jnp.int32)])
    args = (z, dinv, epack) if weighted else (z, epack)
    return pl.kernel(
        body,
        out_type=jax.ShapeDtypeStruct((kp, 256), jnp.float32),
        mesh=plsc.VectorSubcoreMesh(core_axis_name="c", subcore_axis_name="s"),
        compiler_params=pltpu.CompilerParams(needs_layout_passes=False),
        scratch_types=scratch,
    )(*args)


def _sc_agg(x, row, col, emask, n, dinv=None):
    kp = (n + 511) // 512 * 512
    E = row.shape[0]
    ep = (E + 2 * CHUNK - 1) // (2 * CHUNK) * (2 * CHUNK)
    xp = jnp.pad(x, ((0, kp - n), (0, 0)))
    rp = jnp.pad(row, (0, ep - E))
    cp = jnp.pad(col, (0, ep - E))
    em = jnp.pad(emask, (0, ep - E))
    dp = None if dinv is None else jnp.pad(dinv, (0, kp - n))
    agg = _seg_fold(xp, rp, cp, em, dp, kp=kp, ep=ep)
    return agg[:n]


def _mm_kernel(x_ref, w_ref, o_ref):
    o_ref[...] = jnp.dot(x_ref[...], w_ref[...])


def _pallas_mm(x, w):
    n, d = x.shape
    npad = (n + 255) // 256 * 256
    xp = jnp.pad(x, ((0, npad - n), (0, 0)))
    out = pl.pallas_call(
        _mm_kernel,
        grid=(npad // 256,),
        in_specs=[pl.BlockSpec((256, d), lambda i: (i, 0)),
                  pl.BlockSpec((d, w.shape[1]), lambda i: (0, 0))],
        out_specs=pl.BlockSpec((256, w.shape[1]), lambda i: (i, 0)),
        out_shape=jax.ShapeDtypeStruct((npad, w.shape[1]), jnp.float32),
    )(xp, w)
    return out[:n]


def _gcn_conv(x, row, col, emask, n, W, b, use_sc=True):
    h = _pallas_mm(x, W)
    if use_sc:
        sl = jnp.arange(n, dtype=row.dtype)
        c = jnp.concatenate([col, sl])
        w = jnp.concatenate([emask, jnp.ones((n,), x.dtype)])
        deg = jnp.zeros((n,), x.dtype).at[c].add(w)
        dinv = jnp.where(deg > 0, 1.0 / jnp.sqrt(jnp.where(deg > 0, deg, 1.0)), 0.0)
        agg = _sc_agg(h, row, col, emask, n, dinv)
        return (agg + (dinv * dinv)[:, None] * h) + b
    sl = jnp.arange(n, dtype=row.dtype)
    r = jnp.concatenate([row, sl])
    c = jnp.concatenate([col, sl])
    w = jnp.concatenate([emask, jnp.ones((n,), x.dtype)])
    deg = jnp.zeros((n,), x.dtype).at[c].add(w)
    dinv = jnp.where(deg > 0, 1.0 / jnp.sqrt(jnp.where(deg > 0, deg, 1.0)), 0.0)
    norm = dinv[r] * w * dinv[c]
    out = jnp.zeros((n, h.shape[1]), x.dtype).at[c].add(norm[:, None] * h[r])
    return out + b


def _graph_conv_score(x, row, col, emask, n, Wrel, brel, Wroot):
    agg = _sc_agg(x, row, col, emask, n)
    return (agg @ Wrel + brel + x @ Wroot).reshape(-1)


def _sag_pool(x, row, col, emask, n, k, Wrel, brel, Wroot):
    attn = _graph_conv_score(x, row, col, emask, n, Wrel, brel, Wroot)
    score = jnp.tanh(attn)
    vals, perm = jax.lax.top_k(score, k)
    x_new = x[perm] * vals[:, None]
    newidx = jnp.full((n,), -1, jnp.int32).at[perm].set(jnp.arange(k, dtype=jnp.int32))
    r2 = newidx[row]
    c2 = newidx[col]
    valid = (r2 >= 0) & (c2 >= 0) & (emask > 0)
    row_new = jnp.where(valid, r2, 0)
    col_new = jnp.where(valid, c2, 0)
    emask_new = valid.astype(x.dtype)
    return x_new, row_new, col_new, emask_new, perm, vals


def _normalize_scores(s):
    mn = s.min(); mx = s.max()
    denom = jnp.where(mx > mn, mx - mn, 1.0)
    return jnp.where(mx > mn, (s - mn) / denom, jnp.ones_like(s) * 0.5)


def kernel(x, edge_index, batch, params):
    row = edge_index[0].astype(jnp.int32)
    col = edge_index[1].astype(jnp.int32)
    emask = jnp.ones((row.shape[0],), x.dtype)
    n_files = N_NODES
    per_layer = []
    cum = jnp.zeros((n_files,), x.dtype)
    xs = []
    k = x.shape[0]
    for i in range(1, 5):
        h = _gcn_conv(x, row, col, emask, k, params['conv%d_W' % i], params['conv%d_b' % i],
                      use_sc=True)
        h = jax.nn.relu(h)
        k_next = int(np.ceil(RATIO * k))
        x, row, col, emask, perm, vals = _sag_pool(h, row, col, emask, k, k_next,
                                                   params['pool%d_Wrel' % i], params['pool%d_brel' % i], params['pool%d_Wroot' % i])
        k = k_next
        xi = jnp.concatenate([x.mean(axis=0), x.max(axis=0)])[None, :]
        xs.append(xi)
        layer_s = jnp.zeros((n_files,), x.dtype).at[perm].set(vals)
        per_layer.append(_normalize_scores(layer_s))
        cum = cum.at[perm].add(vals)
    xsum = xs[0] + xs[1] + xs[2] + xs[3]
    h = jax.nn.relu(xsum @ params['lin1_W'] + params['lin1_b'])
    h = jax.nn.relu(h @ params['lin2_W'] + params['lin2_b'])
    logits = h @ params['lin3_W'] + params['lin3_b']
    probs = jax.nn.softmax(logits, axis=1)
    return logits, probs, jnp.stack(per_layer), _normalize_scores(cum)


# speedup vs baseline: 2.2201x; 1.0001x over previous
import functools
import jax
import jax.numpy as jnp
import numpy as np
from jax import lax
from jax.experimental import pallas as pl
from jax.experimental.pallas import tpu as pltpu, tpu_sc as plsc

N_NODES = 10000
RATIO = 0.5

NW = 32
BATCH = 128
CHUNK = 1024


def _fold_body(kp, ep, weighted, *refs):
    if weighted:
        (z_hbm, dinv_hbm, epack_hbm, agg_hbm,
         ebuf0, ebuf1, pend_p, pend_g, pend_l, gbuf, acc, dinvb, drb,
         sem_e0, sem_e1, sem_g, sem_d) = refs
    else:
        (z_hbm, epack_hbm, agg_hbm,
         ebuf0, ebuf1, pend_p, pend_g, pend_l, gbuf, acc,
         sem_e0, sem_e1, sem_g, sem_d) = refs
    R = kp // NW
    cid = lax.axis_index("c")
    sid = lax.axis_index("s")
    wid = sid * 2 + cid
    base = wid * R

    zero16 = jnp.zeros((16,), jnp.float32)

    def _z(r, _):
        for gg in range(16):
            acc[r, pl.ds(gg * 16, 16)] = zero16
        return 0
    lax.fori_loop(0, R + 1, _z, 0)
    if weighted:
        pltpu.sync_copy(dinv_hbm.at[pl.ds(base, R)], dinvb.at[pl.ds(0, R)])

    def add_batch(n):
        def add_row(j, _):
            l = pend_l[pl.ds(j, 16)][0]
            if weighted:
                dr = drb[pl.ds(j, 16)][0]
                dc = dinvb[pl.ds(l, 16)][0]
                w16 = lax.broadcast(dr * dc, (16,))
                for gg in range(16):
                    plsc.addupdate(acc.at[l, pl.ds(gg * 16, 16)],
                                   gbuf[j, pl.ds(gg * 16, 16)] * w16)
            else:
                for gg in range(16):
                    plsc.addupdate(acc.at[l, pl.ds(gg * 16, 16)],
                                   gbuf[j, pl.ds(gg * 16, 16)])
            return 0
        lax.fori_loop(0, n, add_row, 0)

    def unpack_pend():
        for gg in range(BATCH // 16):
            v = pend_p[pl.ds(gg * 16, 16)]
            pend_g[pl.ds(gg * 16, 16)] = lax.shift_right_logical(v, 9)
            pend_l[pl.ds(gg * 16, 16)] = v & 511

    def fire(cnt):
        unpack_pend()
        h1 = pltpu.async_copy(z_hbm.at[pend_g.at[pl.ds(0, BATCH)]], gbuf, sem_g)
        if weighted:
            h2 = pltpu.async_copy(dinv_hbm.at[pend_g.at[pl.ds(0, BATCH)]], drb.at[pl.ds(0, BATCH)], sem_d)
        h1.wait()
        if weighted:
            h2.wait()
        add_batch(BATCH)
        pend_p[pl.ds(0, 16)] = pend_p[pl.ds(BATCH, 16)]
        return cnt - BATCH

    def scan(ebuf, cnt):
        def grp_body(g, cnt):
            rvec = ebuf[pl.ds(g * 16, 16)]
            cvec = ebuf[pl.ds(CHUNK + g * 16, 16)]
            ivec = ebuf[pl.ds(2 * CHUNK + g * 16, 16)]
            pred = (cvec >= base) & (cvec < base + R) & (ivec > 0)
            lane = lax.broadcasted_iota(jnp.int32, (16,), 0)
            key = jnp.where(pred, lane, jnp.full((16,), 999, jnp.int32))
            packed = lax.shift_left(rvec, 9) | (cvec - base)
            _, sp = plsc.sort_key_val(key, packed)
            pend_p[pl.ds(cnt, 16)] = sp
            cnt = cnt + plsc.all_reduce_population_count(pred)[0]
            return lax.cond(cnt >= BATCH, fire, lambda c: c, cnt)
        return lax.fori_loop(0, CHUNK // 16, grp_body, cnt)

    nchunks = ep // CHUNK
    C3 = 3 * CHUNK
    pltpu.async_copy(epack_hbm.at[pl.ds(0, C3)], ebuf0, sem_e0)
    pltpu.async_copy(epack_hbm.at[pl.ds(C3, C3)], ebuf1, sem_e1)

    def pair_body(p, cnt):
        c0 = 2 * p
        pltpu.make_async_copy(epack_hbm.at[pl.ds(0, C3)], ebuf0, sem_e0).wait()
        cnt = scan(ebuf0, cnt)
        pltpu.async_copy(epack_hbm.at[pl.ds((c0 + 2) * C3, C3)], ebuf0, sem_e0)
        pltpu.make_async_copy(epack_hbm.at[pl.ds(0, C3)], ebuf1, sem_e1).wait()
        cnt = scan(ebuf1, cnt)
        pltpu.async_copy(epack_hbm.at[pl.ds((c0 + 3) * C3, C3)], ebuf1, sem_e1)
        return cnt

    cnt = lax.fori_loop(0, nchunks // 2, pair_body, jnp.int32(0))
    pltpu.make_async_copy(epack_hbm.at[pl.ds(0, C3)], ebuf0, sem_e0).wait()
    pltpu.make_async_copy(epack_hbm.at[pl.ds(0, C3)], ebuf1, sem_e1).wait()

    padp = jnp.full((16,), base * 512 + R, jnp.int32)
    for j in range(BATCH // 16):
        pend_p[pl.ds(cnt + j * 16, 16)] = padp
    unpack_pend()
    h1 = pltpu.async_copy(z_hbm.at[pend_g.at[pl.ds(0, BATCH)]], gbuf, sem_g)
    if weighted:
        h2 = pltpu.async_copy(dinv_hbm.at[pend_g.at[pl.ds(0, BATCH)]], drb.at[pl.ds(0, BATCH)], sem_d)
    h1.wait()
    if weighted:
        h2.wait()
    add_batch(cnt)

    pltpu.sync_copy(acc.at[pl.ds(0, R)], agg_hbm.at[pl.ds(base, R)])


def _seg_fold(z, row, col, em, dinv=None, *, kp, ep):
    R = kp // NW
    weighted = dinv is not None
    body = functools.partial(_fold_body, kp, ep, weighted)
    scratch = [
        pltpu.VMEM((3 * CHUNK,), jnp.int32),
        pltpu.VMEM((3 * CHUNK,), jnp.int32),
        pltpu.VMEM((BATCH + 32,), jnp.int32),
        pltpu.VMEM((BATCH + 16,), jnp.int32),
        pltpu.VMEM((BATCH + 16,), jnp.int32),
        pltpu.VMEM((BATCH, 256), jnp.float32),
        pltpu.VMEM(((R + 1), 256), jnp.float32),
    ]
    if weighted:
        scratch += [pltpu.VMEM((R + 16,), jnp.float32), pltpu.VMEM((BATCH + 16,), jnp.float32)]
    scratch += [pltpu.SemaphoreType.DMA, pltpu.SemaphoreType.DMA,
                pltpu.SemaphoreType.DMA, pltpu.SemaphoreType.DMA]
    nchunks = ep // CHUNK
    epack = jnp.stack([row.reshape(nchunks, CHUNK),
                       col.reshape(nchunks, CHUNK),
                       jax.lax.bitcast_convert_type(em, jnp.int32).reshape(nchunks, CHUNK)],
                      axis=1).reshape(-1)
    epack = jnp.concatenate([epack, jnp.zeros((2 * 3 * CHUNK,), jnp.int32)])
    args = (z, dinv, epack) if weighted else (z, epack)
    return pl.kernel(
        body,
        out_type=jax.ShapeDtypeStruct((kp, 256), jnp.float32),
        mesh=plsc.VectorSubcoreMesh(core_axis_name="c", subcore_axis_name="s"),
        compiler_params=pltpu.CompilerParams(needs_layout_passes=False),
        scratch_types=scratch,
    )(*args)


def _sc_agg(x, row, col, emask, n, dinv=None):
    kp = (n + 511) // 512 * 512
    E = row.shape[0]
    ep = (E + 2 * CHUNK - 1) // (2 * CHUNK) * (2 * CHUNK)
    xp = jnp.pad(x, ((0, kp - n), (0, 0)))
    rp = jnp.pad(row, (0, ep - E))
    cp = jnp.pad(col, (0, ep - E))
    em = jnp.pad(emask, (0, ep - E))
    dp = None if dinv is None else jnp.pad(dinv, (0, kp - n))
    agg = _seg_fold(xp, rp, cp, em, dp, kp=kp, ep=ep)
    return agg[:n]


def _mm_kernel(x_ref, w_ref, o_ref):
    o_ref[...] = jnp.dot(x_ref[...], w_ref[...])


def _pallas_mm(x, w):
    n, d = x.shape
    npad = (n + 255) // 256 * 256
    xp = jnp.pad(x, ((0, npad - n), (0, 0)))
    out = pl.pallas_call(
        _mm_kernel,
        grid=(npad // 256,),
        in_specs=[pl.BlockSpec((256, d), lambda i: (i, 0)),
                  pl.BlockSpec((d, w.shape[1]), lambda i: (0, 0))],
        out_specs=pl.BlockSpec((256, w.shape[1]), lambda i: (i, 0)),
        out_shape=jax.ShapeDtypeStruct((npad, w.shape[1]), jnp.float32),
    )(xp, w)
    return out[:n]


def _gcn_conv(x, row, col, emask, n, W, b):
    h = _pallas_mm(x, W)
    sl = jnp.arange(n, dtype=row.dtype)
    c = jnp.concatenate([col, sl])
    w = jnp.concatenate([emask, jnp.ones((n,), x.dtype)])
    deg = jnp.zeros((n,), x.dtype).at[c].add(w)
    dinv = jnp.where(deg > 0, 1.0 / jnp.sqrt(jnp.where(deg > 0, deg, 1.0)), 0.0)
    agg = _sc_agg(h, row, col, emask, n, dinv)
    return (agg + (dinv * dinv)[:, None] * h) + b


def _graph_conv_score(x, row, col, emask, n, Wrel, brel, Wroot):
    agg = _sc_agg(x, row, col, emask, n)
    return (agg @ Wrel + brel + x @ Wroot).reshape(-1)


def _sag_pool(x, row, col, emask, n, k, Wrel, brel, Wroot):
    attn = _graph_conv_score(x, row, col, emask, n, Wrel, brel, Wroot)
    score = jnp.tanh(attn)
    vals, perm = jax.lax.top_k(score, k)
    x_new = x[perm] * vals[:, None]
    newidx = jnp.full((n,), -1, jnp.int32).at[perm].set(jnp.arange(k, dtype=jnp.int32))
    r2 = newidx[row]
    c2 = newidx[col]
    valid = (r2 >= 0) & (c2 >= 0) & (emask > 0)
    row_new = jnp.where(valid, r2, 0)
    col_new = jnp.where(valid, c2, 0)
    emask_new = valid.astype(x.dtype)
    return x_new, row_new, col_new, emask_new, perm, vals


def _normalize_scores(s):
    mn = s.min(); mx = s.max()
    denom = jnp.where(mx > mn, mx - mn, 1.0)
    return jnp.where(mx > mn, (s - mn) / denom, jnp.ones_like(s) * 0.5)


def kernel(x, edge_index, batch, params):
    row = edge_index[0].astype(jnp.int32)
    col = edge_index[1].astype(jnp.int32)
    emask = jnp.ones((row.shape[0],), x.dtype)
    n_files = N_NODES
    per_layer = []
    cum = jnp.zeros((n_files,), x.dtype)
    xs = []
    k = x.shape[0]
    for i in range(1, 5):
        h = _gcn_conv(x, row, col, emask, k, params['conv%d_W' % i], params['conv%d_b' % i])
        h = jax.nn.relu(h)
        k_next = int(np.ceil(RATIO * k))
        x, row, col, emask, perm, vals = _sag_pool(h, row, col, emask, k, k_next,
                                                   params['pool%d_Wrel' % i], params['pool%d_brel' % i], params['pool%d_Wroot' % i])
        k = k_next
        xi = jnp.concatenate([x.mean(axis=0), x.max(axis=0)])[None, :]
        xs.append(xi)
        layer_s = jnp.zeros((n_files,), x.dtype).at[perm].set(vals)
        per_layer.append(_normalize_scores(layer_s))
        cum = cum.at[perm].add(vals)
    xsum = xs[0] + xs[1] + xs[2] + xs[3]
    h = jax.nn.relu(xsum @ params['lin1_W'] + params['lin1_b'])
    h = jax.nn.relu(h @ params['lin2_W'] + params['lin2_b'])
    logits = h @ params['lin3_W'] + params['lin3_b']
    probs = jax.nn.softmax(logits, axis=1)
    return logits, probs, jnp.stack(per_layer), _normalize_scores(cum)
